# Initial kernel scaffold; baseline (speedup 1.0000x reference)
import jax
import jax.numpy as jnp
from jax.experimental import pallas as pl


def kernel(x, eq_edges, col_edges, cit_edges, collab_edges, sim_edges, Ws1, Wn1, b1, Ws2, Wn2, b2, Wa, ba, Wl1, bl1, Wl2, bl2):
    raise NotImplementedError("write your pallas kernel here")


# speedup vs baseline: 3.9600x; 3.9600x over previous
import functools

import jax
import jax.numpy as jnp
from jax import lax
from jax.experimental import pallas as pl
from jax.experimental.pallas import tpu as pltpu
from jax.experimental.pallas import tpu_sc as plsc

_NC = 2
_NS = 16
_NW = _NC * _NS


def _vmesh():
    return plsc.VectorSubcoreMesh(core_axis_name="c", subcore_axis_name="s")


def _sc_degrees(dst_list, zeros_nh, ones_ch):
    G = len(dst_list)
    E = dst_list[0].shape[0]
    N, Hd = zeros_nh.shape
    C = ones_ch.shape[0]
    ep = E // _NW
    nchunk = ep // C
    rpt = (N // _NS) // 8 * 8
    rem = N - rpt * _NS

    @functools.partial(
        pl.kernel,
        out_type=jax.ShapeDtypeStruct((_NC, G, N, Hd), jnp.float32),
        mesh=_vmesh(),
        scratch_types=[
            pltpu.VMEM((C,), jnp.int32),
            pltpu.VMEM((C, Hd), jnp.float32),
            pltpu.VMEM_SHARED((N, Hd), jnp.float32),
        ],
    )
    def k(d0_hbm, d1_hbm, d2_hbm, d3_hbm, zeros_hbm, ones_hbm, out_hbm,
          idx_v, ones_v, acc):
        cid = lax.axis_index("c")
        sid = lax.axis_index("s")
        wid = cid * _NS + sid
        dst_hbms = [d0_hbm, d1_hbm, d2_hbm, d3_hbm]
        pltpu.sync_copy(ones_hbm, ones_v)
        r0 = sid * rpt
        base = wid * ep
        for g in range(G):
            pltpu.sync_copy(zeros_hbm.at[pl.ds(r0, rpt)],
                            acc.at[pl.ds(r0, rpt)])

            @pl.when(sid == _NS - 1)
            def _():
                pltpu.sync_copy(zeros_hbm.at[pl.ds(rpt * _NS, rem)],
                                acc.at[pl.ds(rpt * _NS, rem)])

            plsc.subcore_barrier()

            @pl.loop(0, nchunk)
            def _(i, g=g):
                off = base + i * C
                pltpu.sync_copy(dst_hbms[g].at[pl.ds(off, C)], idx_v)
                pltpu.sync_copy(ones_v, acc.at[idx_v], add=True)

            plsc.subcore_barrier()
            pltpu.sync_copy(acc.at[pl.ds(r0, rpt)],
                            out_hbm.at[cid, g, pl.ds(r0, rpt)])

            @pl.when(sid == _NS - 1)
            def _():
                pltpu.sync_copy(acc.at[pl.ds(rpt * _NS, rem)],
                                out_hbm.at[cid, g, pl.ds(rpt * _NS, rem)])

    return k(dst_list[0], dst_list[1], dst_list[2], dst_list[3],
             zeros_nh, ones_ch)


def _sc_segsum(table, src, dst, zeros_nh):
    N, Hd = table.shape
    E = src.shape[0]
    C = 200
    ep = E // _NW
    nchunk = ep // C
    rpt = (N // _NS) // 8 * 8
    rem = N - rpt * _NS

    @functools.partial(
        pl.kernel,
        out_type=jax.ShapeDtypeStruct((_NC, N, Hd), jnp.float32),
        mesh=_vmesh(),
        scratch_types=[
            pltpu.VMEM((C,), jnp.int32),
            pltpu.VMEM((C,), jnp.int32),
            pltpu.VMEM((C, Hd), jnp.float32),
            pltpu.VMEM_SHARED((N, Hd), jnp.float32),
        ],
    )
    def k(table_hbm, src_hbm, dst_hbm, zeros_hbm, out_hbm,
          src_v, dst_v, rows_v, acc_sh):
        cid = lax.axis_index("c")
        sid = lax.axis_index("s")
        wid = cid * _NS + sid
        r0 = sid * rpt
        pltpu.sync_copy(zeros_hbm.at[pl.ds(r0, rpt)], acc_sh.at[pl.ds(r0, rpt)])

        @pl.when(sid == _NS - 1)
        def _():
            pltpu.sync_copy(zeros_hbm.at[pl.ds(rpt * _NS, rem)],
                            acc_sh.at[pl.ds(rpt * _NS, rem)])

        plsc.subcore_barrier()
        base = wid * ep

        @pl.loop(0, nchunk)
        def _(i):
            off = base + i * C
            pltpu.sync_copy(src_hbm.at[pl.ds(off, C)], src_v)
            pltpu.sync_copy(dst_hbm.at[pl.ds(off, C)], dst_v)
            pltpu.sync_copy(table_hbm.at[src_v], rows_v)
            pltpu.sync_copy(rows_v, acc_sh.at[dst_v], add=True)

        plsc.subcore_barrier()
        pltpu.sync_copy(acc_sh.at[pl.ds(r0, rpt)],
                        out_hbm.at[cid, pl.ds(r0, rpt)])

        @pl.when(sid == _NS - 1)
        def _():
            pltpu.sync_copy(acc_sh.at[pl.ds(rpt * _NS, rem)],
                            out_hbm.at[cid, pl.ds(rpt * _NS, rem)])

    return k(table, src, dst, zeros_nh)


def _sc_gather2(a, b, src, dst):
    N, Hd = a.shape
    E = src.shape[0]
    C = 400
    ep = E // _NW
    nchunk = ep // C

    @functools.partial(
        pl.kernel,
        out_type=[jax.ShapeDtypeStruct((E, Hd), jnp.float32),
                  jax.ShapeDtypeStruct((E, Hd), jnp.float32)],
        mesh=_vmesh(),
        scratch_types=[
            pltpu.VMEM((C,), jnp.int32),
            pltpu.VMEM((C,), jnp.int32),
            pltpu.VMEM((C, Hd), jnp.float32),
            pltpu.VMEM((C, Hd), jnp.float32),
        ],
    )
    def k(a_hbm, b_hbm, src_hbm, dst_hbm, oa_hbm, ob_hbm,
          src_v, dst_v, ar_v, br_v):
        cid = lax.axis_index("c")
        sid = lax.axis_index("s")
        wid = cid * _NS + sid
        base = wid * ep

        @pl.loop(0, nchunk)
        def _(i):
            off = base + i * C
            pltpu.sync_copy(src_hbm.at[pl.ds(off, C)], src_v)
            pltpu.sync_copy(dst_hbm.at[pl.ds(off, C)], dst_v)
            pltpu.sync_copy(a_hbm.at[src_v], ar_v)
            pltpu.sync_copy(b_hbm.at[dst_v], br_v)
            pltpu.sync_copy(ar_v, oa_hbm.at[pl.ds(off, C)])
            pltpu.sync_copy(br_v, ob_hbm.at[pl.ds(off, C)])

    return k(a, b, src, dst)


def _tc_layer(hin, p0, p1, d0, d1, ws, wn, b, relu):
    N, Hd = hin.shape
    BN = 400
    grid = N // BN

    def body(h_ref, p0_ref, p1_ref, d0_ref, d1_ref, ws_ref, wn_ref, b_ref,
             o_ref):
        deg = d0_ref[:, 0:1] + d1_ref[:, 0:1]
        mean = (p0_ref[...] + p1_ref[...]) / jnp.maximum(deg, 1.0)
        acc = jnp.dot(h_ref[...], ws_ref[...],
                      preferred_element_type=jnp.float32)
        acc = acc + jnp.dot(mean, wn_ref[...],
                            preferred_element_type=jnp.float32)
        acc = acc + b_ref[...]
        if relu:
            acc = jnp.maximum(acc, 0.0)
        o_ref[...] = acc

    row = lambda i: (i, 0)
    full = lambda i: (0, 0)
    return pl.pallas_call(
        body,
        grid=(grid,),
        in_specs=[
            pl.BlockSpec((BN, Hd), row),
            pl.BlockSpec((BN, Hd), row),
            pl.BlockSpec((BN, Hd), row),
            pl.BlockSpec((BN, Hd), row),
            pl.BlockSpec((BN, Hd), row),
            pl.BlockSpec((Hd, Hd), full),
            pl.BlockSpec((Hd, Hd), full),
            pl.BlockSpec((1, Hd), full),
        ],
        out_specs=pl.BlockSpec((BN, Hd), row),
        out_shape=jax.ShapeDtypeStruct((N, Hd), jnp.float32),
    )(hin, p0, p1, d0, d1, ws, wn, b.reshape(1, Hd))


def _tc_fuse(hs, wa, ba, wl1, bl1):
    N, Hd = hs[0].shape
    BN = 400
    grid = N // BN

    def body(h0_ref, h1_ref, h2_ref, h3_ref, wa_ref, ba_ref, wla_ref,
             wlb_ref, bl1_ref, oa_ref, ob_ref):
        hb = [h0_ref[...], h1_ref[...], h2_ref[...], h3_ref[...]]
        logits = [
            jnp.sum(hb[g] * wa_ref[g:g + 1, :], axis=1, keepdims=True)
            + ba_ref[0, g]
            for g in range(4)
        ]
        lg = jnp.concatenate(logits, axis=1)
        m = jnp.max(lg, axis=1, keepdims=True)
        w = jnp.exp(lg - m)
        w = w / jnp.sum(w, axis=1, keepdims=True)
        hsum = hb[0] * w[:, 0:1]
        for g in range(1, 4):
            hsum = hsum + hb[g] * w[:, g:g + 1]
        oa_ref[...] = jnp.dot(hsum, wla_ref[...],
                              preferred_element_type=jnp.float32) + bl1_ref[...]
        ob_ref[...] = jnp.dot(hsum, wlb_ref[...],
                              preferred_element_type=jnp.float32)

    row = lambda i: (i, 0)
    full = lambda i: (0, 0)
    return pl.pallas_call(
        body,
        grid=(grid,),
        in_specs=[
            pl.BlockSpec((BN, Hd), row),
            pl.BlockSpec((BN, Hd), row),
            pl.BlockSpec((BN, Hd), row),
            pl.BlockSpec((BN, Hd), row),
            pl.BlockSpec((4, Hd), full),
            pl.BlockSpec((1, 4), full),
            pl.BlockSpec((Hd, Hd), full),
            pl.BlockSpec((Hd, Hd), full),
            pl.BlockSpec((1, Hd), full),
        ],
        out_specs=[pl.BlockSpec((BN, Hd), row), pl.BlockSpec((BN, Hd), row)],
        out_shape=[jax.ShapeDtypeStruct((N, Hd), jnp.float32),
                   jax.ShapeDtypeStruct((N, Hd), jnp.float32)],
    )(hs[0], hs[1], hs[2], hs[3], wa, ba.reshape(1, 4),
      wl1[:Hd], wl1[Hd:], bl1.reshape(1, Hd))


def _tc_score(ar, br, wl2, bl2):
    E, Hd = ar.shape
    BE = 3200
    grid = E // BE

    def body(a_ref, b_ref, w_ref, bl2_ref, o_ref):
        e = jnp.maximum(a_ref[...] + b_ref[...], 0.0)
        s = jnp.sum(e * w_ref[...], axis=1, keepdims=True) + bl2_ref[0, 0]
        o_ref[...] = 1.0 / (1.0 + jnp.exp(-s))

    out = pl.pallas_call(
        body,
        grid=(grid,),
        in_specs=[
            pl.BlockSpec((BE, Hd), lambda i: (i, 0)),
            pl.BlockSpec((BE, Hd), lambda i: (i, 0)),
            pl.BlockSpec((1, Hd), lambda i: (0, 0)),
            pl.BlockSpec((1, 1), lambda i: (0, 0)),
        ],
        out_specs=pl.BlockSpec((BE, 1), lambda i: (i, 0)),
        out_shape=jax.ShapeDtypeStruct((E, 1), jnp.float32),
    )(ar, br, wl2.reshape(1, Hd), bl2.reshape(1, 1))
    return out.reshape(E)


def kernel(x, eq_edges, col_edges, cit_edges, collab_edges, sim_edges,
           Ws1, Wn1, b1, Ws2, Wn2, b2, Wa, ba, Wl1, bl1, Wl2, bl2):
    N, D = x.shape
    graphs = [eq_edges, col_edges, cit_edges, collab_edges]
    dsts = [g[1] for g in graphs]
    zeros_nh = jnp.zeros((N, D), jnp.float32)
    ones_ch = jnp.ones((200, D), jnp.float32)

    degp = _sc_degrees(dsts, zeros_nh, ones_ch)


    hs = []
    for g in range(4):
        src, dst = graphs[g][0], graphs[g][1]
        p = _sc_segsum(x, src, dst, zeros_nh)
        h1 = _tc_layer(x, p[0], p[1], degp[0, g], degp[1, g],
                       Ws1[g], Wn1[g], b1[g], relu=True)
        p2 = _sc_segsum(h1, src, dst, zeros_nh)
        h2 = _tc_layer(h1, p2[0], p2[1], degp[0, g], degp[1, g],
                       Ws2[g], Wn2[g], b2[g], relu=False)
        hs.append(h2)

    a, b = _tc_fuse(hs, Wa, ba, Wl1, bl1)
    ar, br = _sc_gather2(a, b, sim_edges[0], sim_edges[1])
    return _tc_score(ar, br, Wl2, bl2)


# speedup vs baseline: 5.0146x; 1.2663x over previous
import functools

import jax
import jax.numpy as jnp
from jax import lax
from jax.experimental import pallas as pl
from jax.experimental.pallas import tpu as pltpu
from jax.experimental.pallas import tpu_sc as plsc

_NC = 2
_NS = 16
_NW = _NC * _NS


def _vmesh():
    return plsc.VectorSubcoreMesh(core_axis_name="c", subcore_axis_name="s")


def _sc_degrees(dst_list, zeros_nh, ones_ch):
    G = len(dst_list)
    E = dst_list[0].shape[0]
    N, Hd = zeros_nh.shape
    C = ones_ch.shape[0]
    ep = E // _NW
    nchunk = ep // C
    rpt = (N // _NS) // 8 * 8
    rem = N - rpt * _NS

    @functools.partial(
        pl.kernel,
        out_type=jax.ShapeDtypeStruct((_NC, G, N, Hd), jnp.float32),
        mesh=_vmesh(),
        scratch_types=[
            pltpu.VMEM((C,), jnp.int32),
            pltpu.VMEM((C, Hd), jnp.float32),
            pltpu.VMEM_SHARED((N, Hd), jnp.float32),
        ],
    )
    def k(d0_hbm, d1_hbm, d2_hbm, d3_hbm, zeros_hbm, ones_hbm, out_hbm,
          idx_v, ones_v, acc):
        cid = lax.axis_index("c")
        sid = lax.axis_index("s")
        wid = cid * _NS + sid
        dst_hbms = [d0_hbm, d1_hbm, d2_hbm, d3_hbm]
        pltpu.sync_copy(ones_hbm, ones_v)
        r0 = sid * rpt
        base = wid * ep
        for g in range(G):
            pltpu.sync_copy(zeros_hbm.at[pl.ds(r0, rpt)],
                            acc.at[pl.ds(r0, rpt)])

            @pl.when(sid == _NS - 1)
            def _():
                pltpu.sync_copy(zeros_hbm.at[pl.ds(rpt * _NS, rem)],
                                acc.at[pl.ds(rpt * _NS, rem)])

            plsc.subcore_barrier()

            @pl.loop(0, nchunk)
            def _(i, g=g):
                off = base + i * C
                pltpu.sync_copy(dst_hbms[g].at[pl.ds(off, C)], idx_v)
                pltpu.sync_copy(ones_v, acc.at[idx_v], add=True)

            plsc.subcore_barrier()
            pltpu.sync_copy(acc.at[pl.ds(r0, rpt)],
                            out_hbm.at[cid, g, pl.ds(r0, rpt)])

            @pl.when(sid == _NS - 1)
            def _():
                pltpu.sync_copy(acc.at[pl.ds(rpt * _NS, rem)],
                                out_hbm.at[cid, g, pl.ds(rpt * _NS, rem)])

    return k(dst_list[0], dst_list[1], dst_list[2], dst_list[3],
             zeros_nh, ones_ch)


def _sc_segsum(table, src, dst, zeros_nh):
    N, Hd = table.shape
    E = src.shape[0]
    C = 192
    ep = E // _NW
    nchunk = ep // C
    tail = ep - nchunk * C
    rpt = (N // _NS) // 8 * 8
    rem = N - rpt * _NS

    @functools.partial(
        pl.kernel,
        out_type=jax.ShapeDtypeStruct((_NC, N, Hd), jnp.float32),
        mesh=_vmesh(),
        scratch_types=[
            pltpu.VMEM((C,), jnp.int32),
            pltpu.VMEM((C,), jnp.int32),
            pltpu.VMEM((C,), jnp.int32),
            pltpu.VMEM((C,), jnp.int32),
            pltpu.VMEM((C, Hd), jnp.float32),
            pltpu.VMEM((C, Hd), jnp.float32),
            pltpu.VMEM((16,), jnp.int32),
            pltpu.VMEM((16,), jnp.int32),
            pltpu.VMEM_SHARED((N, Hd), jnp.float32),
            pltpu.SemaphoreType.DMA,
            pltpu.SemaphoreType.DMA,
        ],
    )
    def k(table_hbm, src_hbm, dst_hbm, zeros_hbm, out_hbm,
          sv0, dv0, sv1, dv1, rows0, rows1, tsv, tdv, acc_sh, sem0, sem1):
        cid = lax.axis_index("c")
        sid = lax.axis_index("s")
        wid = cid * _NS + sid
        r0 = sid * rpt
        pltpu.sync_copy(zeros_hbm.at[pl.ds(r0, rpt)], acc_sh.at[pl.ds(r0, rpt)])

        @pl.when(sid == _NS - 1)
        def _():
            pltpu.sync_copy(zeros_hbm.at[pl.ds(rpt * _NS, rem)],
                            acc_sh.at[pl.ds(rpt * _NS, rem)])

        plsc.subcore_barrier()
        base = wid * ep

        def idx_load(off, sv, dv):
            pltpu.sync_copy(src_hbm.at[pl.ds(off, C)], sv)
            pltpu.sync_copy(dst_hbm.at[pl.ds(off, C)], dv)

        def gather(sv, rows, sem):
            pltpu.make_async_copy(table_hbm.at[sv], rows, sem).start()

        def gwait(sv, rows, sem):
            pltpu.make_async_copy(table_hbm.at[sv], rows, sem).wait()

        idx_load(base, sv0, dv0)
        gather(sv0, rows0, sem0)

        @pl.loop(0, nchunk // 2)
        def _(j):
            off1 = base + (2 * j + 1) * C
            idx_load(off1, sv1, dv1)
            gather(sv1, rows1, sem1)
            gwait(sv0, rows0, sem0)
            pltpu.sync_copy(rows0, acc_sh.at[dv0], add=True)

            @pl.when(j + 1 < nchunk // 2)
            def _():
                off0 = base + (2 * j + 2) * C
                idx_load(off0, sv0, dv0)
                gather(sv0, rows0, sem0)

            gwait(sv1, rows1, sem1)
            pltpu.sync_copy(rows1, acc_sh.at[dv1], add=True)

        if tail:
            toff = base + nchunk * C
            pltpu.sync_copy(src_hbm.at[pl.ds(toff, tail)], tsv)
            pltpu.sync_copy(dst_hbm.at[pl.ds(toff, tail)], tdv)
            pltpu.sync_copy(table_hbm.at[tsv], rows0.at[pl.ds(0, tail)])
            pltpu.sync_copy(rows0.at[pl.ds(0, tail)], acc_sh.at[tdv], add=True)

        plsc.subcore_barrier()
        pltpu.sync_copy(acc_sh.at[pl.ds(r0, rpt)],
                        out_hbm.at[cid, pl.ds(r0, rpt)])

        @pl.when(sid == _NS - 1)
        def _():
            pltpu.sync_copy(acc_sh.at[pl.ds(rpt * _NS, rem)],
                            out_hbm.at[cid, pl.ds(rpt * _NS, rem)])

    return k(table, src, dst, zeros_nh)


def _sc_gather2(a, b, src, dst):
    N, Hd = a.shape
    E = src.shape[0]
    C = 400
    ep = E // _NW
    nchunk = ep // C

    @functools.partial(
        pl.kernel,
        out_type=[jax.ShapeDtypeStruct((E, Hd), jnp.float32),
                  jax.ShapeDtypeStruct((E, Hd), jnp.float32)],
        mesh=_vmesh(),
        scratch_types=[
            pltpu.VMEM((C,), jnp.int32),
            pltpu.VMEM((C,), jnp.int32),
            pltpu.VMEM((C, Hd), jnp.float32),
            pltpu.VMEM((C, Hd), jnp.float32),
        ],
    )
    def k(a_hbm, b_hbm, src_hbm, dst_hbm, oa_hbm, ob_hbm,
          src_v, dst_v, ar_v, br_v):
        cid = lax.axis_index("c")
        sid = lax.axis_index("s")
        wid = cid * _NS + sid
        base = wid * ep

        @pl.loop(0, nchunk)
        def _(i):
            off = base + i * C
            pltpu.sync_copy(src_hbm.at[pl.ds(off, C)], src_v)
            pltpu.sync_copy(dst_hbm.at[pl.ds(off, C)], dst_v)
            pltpu.sync_copy(a_hbm.at[src_v], ar_v)
            pltpu.sync_copy(b_hbm.at[dst_v], br_v)
            pltpu.sync_copy(ar_v, oa_hbm.at[pl.ds(off, C)])
            pltpu.sync_copy(br_v, ob_hbm.at[pl.ds(off, C)])

    return k(a, b, src, dst)


def _tc_layer(hin, p0, p1, d0, d1, ws, wn, b, relu):
    N, Hd = hin.shape
    BN = 400
    grid = N // BN

    def body(h_ref, p0_ref, p1_ref, d0_ref, d1_ref, ws_ref, wn_ref, b_ref,
             o_ref):
        deg = d0_ref[:, 0:1] + d1_ref[:, 0:1]
        mean = (p0_ref[...] + p1_ref[...]) / jnp.maximum(deg, 1.0)
        acc = jnp.dot(h_ref[...], ws_ref[...],
                      preferred_element_type=jnp.float32)
        acc = acc + jnp.dot(mean, wn_ref[...],
                            preferred_element_type=jnp.float32)
        acc = acc + b_ref[...]
        if relu:
            acc = jnp.maximum(acc, 0.0)
        o_ref[...] = acc

    row = lambda i: (i, 0)
    full = lambda i: (0, 0)
    return pl.pallas_call(
        body,
        grid=(grid,),
        in_specs=[
            pl.BlockSpec((BN, Hd), row),
            pl.BlockSpec((BN, Hd), row),
            pl.BlockSpec((BN, Hd), row),
            pl.BlockSpec((BN, Hd), row),
            pl.BlockSpec((BN, Hd), row),
            pl.BlockSpec((Hd, Hd), full),
            pl.BlockSpec((Hd, Hd), full),
            pl.BlockSpec((1, Hd), full),
        ],
        out_specs=pl.BlockSpec((BN, Hd), row),
        out_shape=jax.ShapeDtypeStruct((N, Hd), jnp.float32),
    )(hin, p0, p1, d0, d1, ws, wn, b.reshape(1, Hd))


def _tc_fuse(hs, wa, ba, wl1, bl1):
    N, Hd = hs[0].shape
    BN = 400
    grid = N // BN

    def body(h0_ref, h1_ref, h2_ref, h3_ref, wa_ref, ba_ref, wla_ref,
             wlb_ref, bl1_ref, oa_ref, ob_ref):
        hb = [h0_ref[...], h1_ref[...], h2_ref[...], h3_ref[...]]
        logits = [
            jnp.sum(hb[g] * wa_ref[g:g + 1, :], axis=1, keepdims=True)
            + ba_ref[0, g]
            for g in range(4)
        ]
        lg = jnp.concatenate(logits, axis=1)
        m = jnp.max(lg, axis=1, keepdims=True)
        w = jnp.exp(lg - m)
        w = w / jnp.sum(w, axis=1, keepdims=True)
        hsum = hb[0] * w[:, 0:1]
        for g in range(1, 4):
            hsum = hsum + hb[g] * w[:, g:g + 1]
        oa_ref[...] = jnp.dot(hsum, wla_ref[...],
                              preferred_element_type=jnp.float32) + bl1_ref[...]
        ob_ref[...] = jnp.dot(hsum, wlb_ref[...],
                              preferred_element_type=jnp.float32)

    row = lambda i: (i, 0)
    full = lambda i: (0, 0)
    return pl.pallas_call(
        body,
        grid=(grid,),
        in_specs=[
            pl.BlockSpec((BN, Hd), row),
            pl.BlockSpec((BN, Hd), row),
            pl.BlockSpec((BN, Hd), row),
            pl.BlockSpec((BN, Hd), row),
            pl.BlockSpec((4, Hd), full),
            pl.BlockSpec((1, 4), full),
            pl.BlockSpec((Hd, Hd), full),
            pl.BlockSpec((Hd, Hd), full),
            pl.BlockSpec((1, Hd), full),
        ],
        out_specs=[pl.BlockSpec((BN, Hd), row), pl.BlockSpec((BN, Hd), row)],
        out_shape=[jax.ShapeDtypeStruct((N, Hd), jnp.float32),
                   jax.ShapeDtypeStruct((N, Hd), jnp.float32)],
    )(hs[0], hs[1], hs[2], hs[3], wa, ba.reshape(1, 4),
      wl1[:Hd], wl1[Hd:], bl1.reshape(1, Hd))


def _tc_score(ar, br, wl2, bl2):
    E, Hd = ar.shape
    BE = 3200
    grid = E // BE

    def body(a_ref, b_ref, w_ref, bl2_ref, o_ref):
        e = jnp.maximum(a_ref[...] + b_ref[...], 0.0)
        s = jnp.sum(e * w_ref[...], axis=1, keepdims=True) + bl2_ref[0, 0]
        o_ref[...] = 1.0 / (1.0 + jnp.exp(-s))

    out = pl.pallas_call(
        body,
        grid=(grid,),
        in_specs=[
            pl.BlockSpec((BE, Hd), lambda i: (i, 0)),
            pl.BlockSpec((BE, Hd), lambda i: (i, 0)),
            pl.BlockSpec((1, Hd), lambda i: (0, 0)),
            pl.BlockSpec((1, 1), lambda i: (0, 0)),
        ],
        out_specs=pl.BlockSpec((BE, 1), lambda i: (i, 0)),
        out_shape=jax.ShapeDtypeStruct((E, 1), jnp.float32),
    )(ar, br, wl2.reshape(1, Hd), bl2.reshape(1, 1))
    return out.reshape(E)


def kernel(x, eq_edges, col_edges, cit_edges, collab_edges, sim_edges,
           Ws1, Wn1, b1, Ws2, Wn2, b2, Wa, ba, Wl1, bl1, Wl2, bl2):
    N, D = x.shape
    graphs = [eq_edges, col_edges, cit_edges, collab_edges]
    dsts = [g[1] for g in graphs]
    zeros_nh = jnp.zeros((N, D), jnp.float32)
    ones_ch = jnp.ones((200, D), jnp.float32)

    degp = _sc_degrees(dsts, zeros_nh, ones_ch)


    hs = []
    for g in range(4):
        src, dst = graphs[g][0], graphs[g][1]
        p = _sc_segsum(x, src, dst, zeros_nh)
        h1 = _tc_layer(x, p[0], p[1], degp[0, g], degp[1, g],
                       Ws1[g], Wn1[g], b1[g], relu=True)
        p2 = _sc_segsum(h1, src, dst, zeros_nh)
        h2 = _tc_layer(h1, p2[0], p2[1], degp[0, g], degp[1, g],
                       Ws2[g], Wn2[g], b2[g], relu=False)
        hs.append(h2)

    a, b = _tc_fuse(hs, Wa, ba, Wl1, bl1)
    ar, br = _sc_gather2(a, b, sim_edges[0], sim_edges[1])
    return _tc_score(ar, br, Wl2, bl2)
